# baseline (speedup 1.0000x reference)
import functools

import jax
import jax.numpy as jnp
from jax import lax
from jax.experimental import pallas as pl
from jax.experimental.pallas import tpu as pltpu
from jax.experimental.pallas import tpu_sc as plsc

N_NODES = 10000
DIM = 64
NUM_CORES = 2
NUM_SUBCORES = 16
NUM_TILES = NUM_CORES * NUM_SUBCORES
CHUNK = 128
E_PAD = 327680
N_PAD = 10240
ROWS_PER_SUB = N_PAD // NUM_SUBCORES

_MESH = plsc.VectorSubcoreMesh(core_axis_name="c", subcore_axis_name="s")
_SC_PARAMS = pltpu.CompilerParams(use_tc_tiling_on_sc=False)


def _sc_degree(col2, ones_pay, zeros16):
    tot = col2.shape[0]
    per_tile = tot // NUM_TILES

    @functools.partial(
        pl.kernel, mesh=_MESH,
        out_type=jax.ShapeDtypeStruct((NUM_CORES, N_PAD, 16), jnp.float32),
        compiler_params=_SC_PARAMS,
        scratch_types=[
            pltpu.VMEM((per_tile, CHUNK), jnp.int32),
            pltpu.VMEM((CHUNK, 16), jnp.float32),
            pltpu.VMEM_SHARED((N_PAD, 16), jnp.float32),
            pltpu.SemaphoreType.DMA,
        ])
    def k(col_hbm, ones_hbm, zeros_hbm, out_hbm, coli_v, ones_v, acc_sh, sem):
        c = lax.axis_index("c")
        s = lax.axis_index("s")
        g = c * NUM_SUBCORES + s
        pltpu.sync_copy(zeros_hbm, acc_sh.at[pl.ds(s * ROWS_PER_SUB, ROWS_PER_SUB)])
        pltpu.sync_copy(ones_hbm, ones_v)
        pltpu.sync_copy(col_hbm.at[pl.ds(g * per_tile, per_tile)], coli_v)
        plsc.subcore_barrier()

        K = 10

        @pl.loop(0, per_tile, step=K)
        def _(i):
            for bq in range(K):
                pltpu.async_copy(ones_v, acc_sh.at[coli_v.at[i + bq]],
                                 sem, add=True)
            for bq in range(K):
                pltpu.make_async_copy(ones_v, acc_sh.at[coli_v.at[i + bq]],
                                      sem).wait()

        plsc.subcore_barrier()
        pltpu.sync_copy(acc_sh.at[pl.ds(s * ROWS_PER_SUB, ROWS_PER_SUB)],
                        out_hbm.at[c, pl.ds(s * ROWS_PER_SUB, ROWS_PER_SUB)])

    return k(col2, ones_pay, zeros16)


def _sc_scatter(u, row2, col2, zeros64):
    tot = row2.shape[0]
    per_tile = tot // NUM_TILES

    @functools.partial(
        pl.kernel, mesh=_MESH,
        out_type=jax.ShapeDtypeStruct((NUM_CORES, N_PAD, DIM), jnp.float32),
        compiler_params=_SC_PARAMS,
        scratch_types=(
            [pltpu.VMEM((per_tile, CHUNK), jnp.int32),
             pltpu.VMEM((per_tile, CHUNK), jnp.int32)]
            + [pltpu.VMEM((CHUNK, DIM), jnp.float32) for _ in range(8)]
            + [pltpu.VMEM_SHARED((N_PAD, DIM), jnp.float32),
               pltpu.SemaphoreType.DMA, pltpu.SemaphoreType.DMA,
               pltpu.SemaphoreType.DMA, pltpu.SemaphoreType.DMA]))
    def k(u_hbm, row_hbm, col_hbm, zeros_hbm, out_hbm,
          rowi_v, coli_v, b0, b1, b2, b3, b4, b5, b6, b7, acc_sh,
          gsem_a, gsem_b, ssem_a, ssem_b):
        c = lax.axis_index("c")
        s = lax.axis_index("s")
        g = c * NUM_SUBCORES + s
        grp_a = [b0, b1, b2, b3]
        grp_b = [b4, b5, b6, b7]
        pltpu.sync_copy(zeros_hbm, acc_sh.at[pl.ds(s * ROWS_PER_SUB, ROWS_PER_SUB)])
        pltpu.sync_copy(row_hbm.at[pl.ds(g * per_tile, per_tile)], rowi_v)
        pltpu.sync_copy(col_hbm.at[pl.ds(g * per_tile, per_tile)], coli_v)
        plsc.subcore_barrier()

        for q in range(4):
            pltpu.async_copy(u_hbm.at[rowi_v.at[q]], grp_a[q], gsem_a)
        for q in range(4):
            pltpu.async_copy(u_hbm.at[rowi_v.at[4 + q]], grp_b[q], gsem_b)

        @pl.loop(0, per_tile, step=8)
        def _(i):
            for q in range(4):
                pltpu.make_async_copy(
                    u_hbm.at[rowi_v.at[i + q]], grp_a[q], gsem_a).wait()
            for q in range(4):
                pltpu.async_copy(grp_a[q], acc_sh.at[coli_v.at[i + q]],
                                 ssem_a, add=True)
            for q in range(4):
                pltpu.make_async_copy(
                    u_hbm.at[rowi_v.at[i + 4 + q]], grp_b[q], gsem_b).wait()
            for q in range(4):
                pltpu.async_copy(grp_b[q], acc_sh.at[coli_v.at[i + 4 + q]],
                                 ssem_b, add=True)
            for q in range(4):
                pltpu.make_async_copy(
                    grp_a[q], acc_sh.at[coli_v.at[i + q]], ssem_a).wait()

            @pl.when(i + 8 < per_tile)
            def _():
                for q in range(4):
                    pltpu.async_copy(
                        u_hbm.at[rowi_v.at[i + 8 + q]], grp_a[q], gsem_a)

            for q in range(4):
                pltpu.make_async_copy(
                    grp_b[q], acc_sh.at[coli_v.at[i + 4 + q]], ssem_b).wait()

            @pl.when(i + 12 < per_tile)
            def _():
                for q in range(4):
                    pltpu.async_copy(
                        u_hbm.at[rowi_v.at[i + 12 + q]], grp_b[q], gsem_b)

        plsc.subcore_barrier()
        pltpu.sync_copy(acc_sh.at[pl.ds(s * ROWS_PER_SUB, ROWS_PER_SUB)],
                        out_hbm.at[c, pl.ds(s * ROWS_PER_SUB, ROWS_PER_SUB)])

    return k(u, row2, col2, zeros64)


def _tc_u(hist, xw):
    bm = 2000

    def body(h_ref, xw_ref, o_ref):
        deg = 1.0 + h_ref[0, :, 0:1] + h_ref[1, :, 0:1]
        o_ref[...] = jax.lax.rsqrt(deg) * xw_ref[...]

    return pl.pallas_call(
        body,
        grid=(N_NODES // bm,),
        in_specs=[pl.BlockSpec((NUM_CORES, bm, 16), lambda i: (0, i, 0)),
                  pl.BlockSpec((bm, DIM), lambda i: (i, 0))],
        out_specs=pl.BlockSpec((bm, DIM), lambda i: (i, 0)),
        out_shape=jax.ShapeDtypeStruct((N_NODES, DIM), jnp.float32),
    )(hist, xw)


def _tc_xw(z, W):
    bm = 2000

    def body(z_ref, w_ref, o_ref):
        o_ref[...] = jax.lax.dot(z_ref[...], w_ref[...],
                                 precision=lax.Precision.HIGHEST,
                                 preferred_element_type=jnp.float32)

    return pl.pallas_call(
        body,
        grid=(N_NODES // bm,),
        in_specs=[pl.BlockSpec((bm, DIM), lambda i: (i, 0)),
                  pl.BlockSpec((DIM, DIM), lambda i: (0, 0))],
        out_specs=pl.BlockSpec((bm, DIM), lambda i: (i, 0)),
        out_shape=jax.ShapeDtypeStruct((N_NODES, DIM), jnp.float32),
    )(z, W)


def _tc_gram_fused(part, hist, xw, b2):
    bm, bn = 1024, 2048
    gi = (N_NODES + bm - 1) // bm
    gj = (N_NODES + bn - 1) // bn

    def body(part_ref, hist_ref, xw_ref, b_ref, o_ref, h_sc):
        i = pl.program_id(0)
        j = pl.program_id(1)

        @pl.when(jnp.logical_and(i == 0, j == 0))
        def _():
            p = part_ref[0, :N_NODES, :] + part_ref[1, :N_NODES, :]
            deg = (1.0 + hist_ref[0, :N_NODES, 0:1]
                   + hist_ref[1, :N_NODES, 0:1])
            dinv = jax.lax.rsqrt(deg)
            h = jnp.maximum(
                dinv * p + dinv * dinv * xw_ref[...] + b_ref[...], 0.0)
            h_sc[:N_NODES, :] = h.astype(jnp.bfloat16)
            h_sc[N_NODES:, :] = jnp.zeros((N_PAD - N_NODES, DIM), jnp.bfloat16)

        a = h_sc[pl.ds(i * bm, bm), :]
        b = h_sc[pl.ds(j * bn, bn), :]
        o_ref[...] = jax.lax.dot_general(
            a, b, (((1,), (1,)), ((), ())),
            preferred_element_type=jnp.float32)

    return pl.pallas_call(
        body,
        grid=(gi, gj),
        in_specs=[
            pl.BlockSpec((NUM_CORES, N_PAD, DIM), lambda i, j: (0, 0, 0)),
            pl.BlockSpec((NUM_CORES, N_PAD, 16), lambda i, j: (0, 0, 0)),
            pl.BlockSpec((N_NODES, DIM), lambda i, j: (0, 0)),
            pl.BlockSpec((1, DIM), lambda i, j: (0, 0)),
        ],
        out_specs=pl.BlockSpec((bm, bn), lambda i, j: (i, j)),
        out_shape=jax.ShapeDtypeStruct((N_NODES, N_NODES), jnp.float32),
        scratch_shapes=[pltpu.VMEM((N_PAD, DIM), jnp.bfloat16)],
    )(part, hist, xw, b2)


def kernel(z, edge_index, W, b):
    e = edge_index.shape[1]
    npad = E_PAD - e
    ei = edge_index.astype(jnp.int32)
    pad = jnp.concatenate(
        [jnp.zeros((1, npad), jnp.int32),
         jnp.full((1, npad), N_PAD - 1, jnp.int32)], axis=0)
    ei = jnp.concatenate([ei, pad], axis=1)
    row = ei[0].reshape(-1, CHUNK)
    col = ei[1].reshape(-1, CHUNK)

    ones_pay = jnp.ones((CHUNK, 16), jnp.float32)
    zeros16 = jnp.zeros((ROWS_PER_SUB, 16), jnp.float32)
    zeros64 = jnp.zeros((ROWS_PER_SUB, DIM), jnp.float32)

    hist = _sc_degree(col, ones_pay, zeros16)
    xw = _tc_xw(z, W)
    u = _tc_u(hist, xw)

    part = _sc_scatter(u, row, col, zeros64)
    return _tc_gram_fused(part, hist, xw, b.reshape(1, DIM))


# speedup vs baseline: 1.5421x; 1.5421x over previous
import functools

import jax
import jax.numpy as jnp
from jax import lax
from jax.experimental import pallas as pl
from jax.experimental.pallas import tpu as pltpu
from jax.experimental.pallas import tpu_sc as plsc

N_NODES = 10000
DIM = 64
NUM_CORES = 2
NUM_SUBCORES = 16
NUM_TILES = NUM_CORES * NUM_SUBCORES
CHUNK = 125
N_PAD = 10240
ROWS_PER_SUB = N_PAD // NUM_SUBCORES

_MESH = plsc.VectorSubcoreMesh(core_axis_name="c", subcore_axis_name="s")
_SC_PARAMS = pltpu.CompilerParams(use_tc_tiling_on_sc=False)


def _sc_degree(col1, ones_pay, zeros16):
    tot = col1.shape[0]
    per_tile = tot // NUM_TILES

    @functools.partial(
        pl.kernel, mesh=_MESH,
        out_type=jax.ShapeDtypeStruct((NUM_CORES, N_PAD, 16), jnp.float32),
        compiler_params=_SC_PARAMS,
        scratch_types=[
            pltpu.VMEM((per_tile, CHUNK), jnp.int32),
            pltpu.VMEM((CHUNK, 16), jnp.float32),
            pltpu.VMEM_SHARED((N_PAD, 16), jnp.float32),
            pltpu.SemaphoreType.DMA,
        ])
    def k(col_hbm, ones_hbm, zeros_hbm, out_hbm, coli_v, ones_v, acc_sh, sem):
        c = lax.axis_index("c")
        s = lax.axis_index("s")
        g = c * NUM_SUBCORES + s
        pltpu.sync_copy(zeros_hbm, acc_sh.at[pl.ds(s * ROWS_PER_SUB, ROWS_PER_SUB)])
        pltpu.sync_copy(ones_hbm, ones_v)
        pltpu.sync_copy(col_hbm.at[pl.ds(g * per_tile, per_tile)], coli_v)
        plsc.subcore_barrier()

        K = 10

        @pl.loop(0, per_tile, step=K)
        def _(i):
            for bq in range(K):
                pltpu.async_copy(ones_v, acc_sh.at[coli_v.at[i + bq]],
                                 sem, add=True)
            for bq in range(K):
                pltpu.make_async_copy(ones_v, acc_sh.at[coli_v.at[i + bq]],
                                      sem).wait()

        plsc.subcore_barrier()
        pltpu.sync_copy(acc_sh.at[pl.ds(s * ROWS_PER_SUB, ROWS_PER_SUB)],
                        out_hbm.at[c, pl.ds(s * ROWS_PER_SUB, ROWS_PER_SUB)])

    return k(col1, ones_pay, zeros16)


def _sc_scatter(u, row1, col1, zeros64):
    tot = row1.shape[0]
    per_tile = tot // NUM_TILES

    @functools.partial(
        pl.kernel, mesh=_MESH,
        out_type=jax.ShapeDtypeStruct((NUM_CORES, N_PAD, DIM), jnp.float32),
        compiler_params=_SC_PARAMS,
        scratch_types=(
            [pltpu.VMEM((per_tile, CHUNK), jnp.int32),
             pltpu.VMEM((per_tile, CHUNK), jnp.int32)]
            + [pltpu.VMEM((CHUNK, DIM), jnp.float32) for _ in range(8)]
            + [pltpu.VMEM_SHARED((N_PAD, DIM), jnp.float32),
               pltpu.SemaphoreType.DMA, pltpu.SemaphoreType.DMA,
               pltpu.SemaphoreType.DMA, pltpu.SemaphoreType.DMA]))
    def k(u_hbm, row_hbm, col_hbm, zeros_hbm, out_hbm,
          rowi_v, coli_v, b0, b1, b2, b3, b4, b5, b6, b7, acc_sh,
          gsem_a, gsem_b, ssem_a, ssem_b):
        c = lax.axis_index("c")
        s = lax.axis_index("s")
        g = c * NUM_SUBCORES + s
        grp_a = [b0, b1, b2, b3]
        grp_b = [b4, b5, b6, b7]
        pltpu.sync_copy(zeros_hbm, acc_sh.at[pl.ds(s * ROWS_PER_SUB, ROWS_PER_SUB)])
        pltpu.sync_copy(row_hbm.at[pl.ds(g * per_tile, per_tile)], rowi_v)
        pltpu.sync_copy(col_hbm.at[pl.ds(g * per_tile, per_tile)], coli_v)
        plsc.subcore_barrier()

        for q in range(4):
            pltpu.async_copy(u_hbm.at[rowi_v.at[q]], grp_a[q], gsem_a)
        for q in range(4):
            pltpu.async_copy(u_hbm.at[rowi_v.at[4 + q]], grp_b[q], gsem_b)

        @pl.loop(0, per_tile, step=8)
        def _(i):
            for q in range(4):
                pltpu.make_async_copy(
                    u_hbm.at[rowi_v.at[i + q]], grp_a[q], gsem_a).wait()
            for q in range(4):
                pltpu.async_copy(grp_a[q], acc_sh.at[coli_v.at[i + q]],
                                 ssem_a, add=True)
            for q in range(4):
                pltpu.make_async_copy(
                    u_hbm.at[rowi_v.at[i + 4 + q]], grp_b[q], gsem_b).wait()
            for q in range(4):
                pltpu.async_copy(grp_b[q], acc_sh.at[coli_v.at[i + 4 + q]],
                                 ssem_b, add=True)
            for q in range(4):
                pltpu.make_async_copy(
                    grp_a[q], acc_sh.at[coli_v.at[i + q]], ssem_a).wait()

            @pl.when(i + 8 < per_tile)
            def _():
                for q in range(4):
                    pltpu.async_copy(
                        u_hbm.at[rowi_v.at[i + 8 + q]], grp_a[q], gsem_a)

            for q in range(4):
                pltpu.make_async_copy(
                    grp_b[q], acc_sh.at[coli_v.at[i + 4 + q]], ssem_b).wait()

            @pl.when(i + 12 < per_tile)
            def _():
                for q in range(4):
                    pltpu.async_copy(
                        u_hbm.at[rowi_v.at[i + 12 + q]], grp_b[q], gsem_b)

        plsc.subcore_barrier()
        pltpu.sync_copy(acc_sh.at[pl.ds(s * ROWS_PER_SUB, ROWS_PER_SUB)],
                        out_hbm.at[c, pl.ds(s * ROWS_PER_SUB, ROWS_PER_SUB)])

    return k(u, row1, col1, zeros64)


def _tc_u(hist, xw):
    bm = 2000

    def body(h_ref, xw_ref, o_ref):
        deg = 1.0 + h_ref[0, :, 0:1] + h_ref[1, :, 0:1]
        o_ref[...] = jax.lax.rsqrt(deg) * xw_ref[...]

    return pl.pallas_call(
        body,
        grid=(N_NODES // bm,),
        in_specs=[pl.BlockSpec((NUM_CORES, bm, 16), lambda i: (0, i, 0)),
                  pl.BlockSpec((bm, DIM), lambda i: (i, 0))],
        out_specs=pl.BlockSpec((bm, DIM), lambda i: (i, 0)),
        out_shape=jax.ShapeDtypeStruct((N_NODES, DIM), jnp.float32),
    )(hist, xw)


def _tc_xw(z, W):
    bm = 2000

    def body(z_ref, w_ref, o_ref):
        o_ref[...] = jax.lax.dot(z_ref[...], w_ref[...],
                                 precision=lax.Precision.HIGHEST,
                                 preferred_element_type=jnp.float32)

    return pl.pallas_call(
        body,
        grid=(N_NODES // bm,),
        in_specs=[pl.BlockSpec((bm, DIM), lambda i: (i, 0)),
                  pl.BlockSpec((DIM, DIM), lambda i: (0, 0))],
        out_specs=pl.BlockSpec((bm, DIM), lambda i: (i, 0)),
        out_shape=jax.ShapeDtypeStruct((N_NODES, DIM), jnp.float32),
    )(z, W)


def _tc_gram_fused(part, hist, xw, b2):
    bm, bn = 1024, 2048
    gi = (N_NODES + bm - 1) // bm
    gj = (N_NODES + bn - 1) // bn

    def body(part_ref, hist_ref, xw_ref, b_ref, o_ref, h_sc):
        i = pl.program_id(0)
        j = pl.program_id(1)

        @pl.when(jnp.logical_and(i == 0, j == 0))
        def _():
            p = part_ref[0, :N_NODES, :] + part_ref[1, :N_NODES, :]
            deg = (1.0 + hist_ref[0, :N_NODES, 0:1]
                   + hist_ref[1, :N_NODES, 0:1])
            dinv = jax.lax.rsqrt(deg)
            h = jnp.maximum(
                dinv * p + dinv * dinv * xw_ref[...] + b_ref[...], 0.0)
            h_sc[:N_NODES, :] = h.astype(jnp.bfloat16)
            h_sc[N_NODES:, :] = jnp.zeros((N_PAD - N_NODES, DIM), jnp.bfloat16)

        a = h_sc[pl.ds(i * bm, bm), :]
        b = h_sc[pl.ds(j * bn, bn), :]
        o_ref[...] = jax.lax.dot_general(
            a, b, (((1,), (1,)), ((), ())),
            preferred_element_type=jnp.float32)

    return pl.pallas_call(
        body,
        grid=(gi, gj),
        in_specs=[
            pl.BlockSpec((NUM_CORES, N_PAD, DIM), lambda i, j: (0, 0, 0)),
            pl.BlockSpec((NUM_CORES, N_PAD, 16), lambda i, j: (0, 0, 0)),
            pl.BlockSpec((N_NODES, DIM), lambda i, j: (0, 0)),
            pl.BlockSpec((1, DIM), lambda i, j: (0, 0)),
        ],
        out_specs=pl.BlockSpec((bm, bn), lambda i, j: (i, j)),
        out_shape=jax.ShapeDtypeStruct((N_NODES, N_NODES), jnp.float32),
        scratch_shapes=[pltpu.VMEM((N_PAD, DIM), jnp.bfloat16)],
    )(part, hist, xw, b2)


def kernel(z, edge_index, W, b):
    row = edge_index[0].astype(jnp.int32).reshape(-1, CHUNK)
    col = edge_index[1].astype(jnp.int32).reshape(-1, CHUNK)

    ones_pay = jnp.ones((CHUNK, 16), jnp.float32)
    zeros16 = jnp.zeros((ROWS_PER_SUB, 16), jnp.float32)
    zeros64 = jnp.zeros((ROWS_PER_SUB, DIM), jnp.float32)

    hist = _sc_degree(col, ones_pay, zeros16)
    xw = _tc_xw(z, W)
    u = _tc_u(hist, xw)

    part = _sc_scatter(u, row, col, zeros64)
    return _tc_gram_fused(part, hist, xw, b.reshape(1, DIM))


# speedup vs baseline: 1.5790x; 1.0239x over previous
import functools

import jax
import jax.numpy as jnp
from jax import lax
from jax.experimental import pallas as pl
from jax.experimental.pallas import tpu as pltpu
from jax.experimental.pallas import tpu_sc as plsc

N_NODES = 10000
DIM = 64
NUM_CORES = 2
NUM_SUBCORES = 16
NUM_TILES = NUM_CORES * NUM_SUBCORES
CHUNK = 125
N_PAD = 10240
ROWS_PER_SUB = N_PAD // NUM_SUBCORES

_MESH = plsc.VectorSubcoreMesh(core_axis_name="c", subcore_axis_name="s")
_SC_PARAMS = pltpu.CompilerParams(use_tc_tiling_on_sc=False)


def _sc_degree(ei3, ones_pay, zeros16):
    tot = ei3.shape[1]
    per_tile = tot // NUM_TILES

    @functools.partial(
        pl.kernel, mesh=_MESH,
        out_type=jax.ShapeDtypeStruct((NUM_CORES, N_PAD, 16), jnp.float32),
        compiler_params=_SC_PARAMS,
        scratch_types=[
            pltpu.VMEM((per_tile, CHUNK), jnp.int32),
            pltpu.VMEM((CHUNK, 16), jnp.float32),
            pltpu.VMEM_SHARED((N_PAD, 16), jnp.float32),
            pltpu.SemaphoreType.DMA,
        ])
    def k(ei_hbm, ones_hbm, zeros_hbm, out_hbm, coli_v, ones_v, acc_sh, sem):
        c = lax.axis_index("c")
        s = lax.axis_index("s")
        g = c * NUM_SUBCORES + s
        pltpu.sync_copy(zeros_hbm, acc_sh.at[pl.ds(s * ROWS_PER_SUB, ROWS_PER_SUB)])
        pltpu.sync_copy(ones_hbm, ones_v)
        pltpu.sync_copy(ei_hbm.at[1, pl.ds(g * per_tile, per_tile)], coli_v)
        plsc.subcore_barrier()

        K = 10

        @pl.loop(0, per_tile, step=K)
        def _(i):
            for bq in range(K):
                pltpu.async_copy(ones_v, acc_sh.at[coli_v.at[i + bq]],
                                 sem, add=True)
            for bq in range(K):
                pltpu.make_async_copy(ones_v, acc_sh.at[coli_v.at[i + bq]],
                                      sem).wait()

        plsc.subcore_barrier()
        pltpu.sync_copy(acc_sh.at[pl.ds(s * ROWS_PER_SUB, ROWS_PER_SUB)],
                        out_hbm.at[c, pl.ds(s * ROWS_PER_SUB, ROWS_PER_SUB)])

    return k(ei3, ones_pay, zeros16)


def _sc_scatter(u, ei3, zeros64):
    tot = ei3.shape[1]
    per_tile = tot // NUM_TILES

    @functools.partial(
        pl.kernel, mesh=_MESH,
        out_type=jax.ShapeDtypeStruct((NUM_CORES, N_PAD, DIM), jnp.float32),
        compiler_params=_SC_PARAMS,
        scratch_types=(
            [pltpu.VMEM((per_tile, CHUNK), jnp.int32),
             pltpu.VMEM((per_tile, CHUNK), jnp.int32)]
            + [pltpu.VMEM((CHUNK, DIM), jnp.float32) for _ in range(8)]
            + [pltpu.VMEM_SHARED((N_PAD, DIM), jnp.float32),
               pltpu.SemaphoreType.DMA, pltpu.SemaphoreType.DMA,
               pltpu.SemaphoreType.DMA, pltpu.SemaphoreType.DMA]))
    def k(u_hbm, ei_hbm, zeros_hbm, out_hbm,
          rowi_v, coli_v, b0, b1, b2, b3, b4, b5, b6, b7, acc_sh,
          gsem_a, gsem_b, ssem_a, ssem_b):
        c = lax.axis_index("c")
        s = lax.axis_index("s")
        g = c * NUM_SUBCORES + s
        grp_a = [b0, b1, b2, b3]
        grp_b = [b4, b5, b6, b7]
        pltpu.sync_copy(zeros_hbm, acc_sh.at[pl.ds(s * ROWS_PER_SUB, ROWS_PER_SUB)])
        pltpu.sync_copy(ei_hbm.at[0, pl.ds(g * per_tile, per_tile)], rowi_v)
        pltpu.sync_copy(ei_hbm.at[1, pl.ds(g * per_tile, per_tile)], coli_v)
        plsc.subcore_barrier()

        for q in range(4):
            pltpu.async_copy(u_hbm.at[rowi_v.at[q]], grp_a[q], gsem_a)
        for q in range(4):
            pltpu.async_copy(u_hbm.at[rowi_v.at[4 + q]], grp_b[q], gsem_b)

        @pl.loop(0, per_tile, step=8)
        def _(i):
            for q in range(4):
                pltpu.make_async_copy(
                    u_hbm.at[rowi_v.at[i + q]], grp_a[q], gsem_a).wait()
            for q in range(4):
                pltpu.async_copy(grp_a[q], acc_sh.at[coli_v.at[i + q]],
                                 ssem_a, add=True)
            for q in range(4):
                pltpu.make_async_copy(
                    u_hbm.at[rowi_v.at[i + 4 + q]], grp_b[q], gsem_b).wait()
            for q in range(4):
                pltpu.async_copy(grp_b[q], acc_sh.at[coli_v.at[i + 4 + q]],
                                 ssem_b, add=True)
            for q in range(4):
                pltpu.make_async_copy(
                    grp_a[q], acc_sh.at[coli_v.at[i + q]], ssem_a).wait()

            @pl.when(i + 8 < per_tile)
            def _():
                for q in range(4):
                    pltpu.async_copy(
                        u_hbm.at[rowi_v.at[i + 8 + q]], grp_a[q], gsem_a)

            for q in range(4):
                pltpu.make_async_copy(
                    grp_b[q], acc_sh.at[coli_v.at[i + 4 + q]], ssem_b).wait()

            @pl.when(i + 12 < per_tile)
            def _():
                for q in range(4):
                    pltpu.async_copy(
                        u_hbm.at[rowi_v.at[i + 12 + q]], grp_b[q], gsem_b)

        plsc.subcore_barrier()
        pltpu.sync_copy(acc_sh.at[pl.ds(s * ROWS_PER_SUB, ROWS_PER_SUB)],
                        out_hbm.at[c, pl.ds(s * ROWS_PER_SUB, ROWS_PER_SUB)])

    return k(u, ei3, zeros64)


def _tc_xw_u(z, W, hist):
    bm = 2000

    def body(z_ref, w_ref, h_ref, xw_ref, u_ref):
        xwv = jax.lax.dot(z_ref[...], w_ref[...],
                          precision=lax.Precision.HIGHEST,
                          preferred_element_type=jnp.float32)
        xw_ref[...] = xwv
        deg = 1.0 + h_ref[0, :, 0:1] + h_ref[1, :, 0:1]
        u_ref[...] = jax.lax.rsqrt(deg) * xwv

    return pl.pallas_call(
        body,
        grid=(N_NODES // bm,),
        in_specs=[pl.BlockSpec((bm, DIM), lambda i: (i, 0)),
                  pl.BlockSpec((DIM, DIM), lambda i: (0, 0)),
                  pl.BlockSpec((NUM_CORES, bm, 16), lambda i: (0, i, 0))],
        out_specs=[pl.BlockSpec((bm, DIM), lambda i: (i, 0)),
                   pl.BlockSpec((bm, DIM), lambda i: (i, 0))],
        out_shape=[jax.ShapeDtypeStruct((N_NODES, DIM), jnp.float32),
                   jax.ShapeDtypeStruct((N_NODES, DIM), jnp.float32)],
    )(z, W, hist)


def _tc_gram_fused(part, hist, xw, b2):
    bm, bn = 1024, 2048
    gi = (N_NODES + bm - 1) // bm
    gj = (N_NODES + bn - 1) // bn

    def body(part_ref, hist_ref, xw_ref, b_ref, o_ref, h_sc):
        i = pl.program_id(0)
        j = pl.program_id(1)

        @pl.when(jnp.logical_and(i == 0, j == 0))
        def _():
            p = part_ref[0, :N_NODES, :] + part_ref[1, :N_NODES, :]
            deg = (1.0 + hist_ref[0, :N_NODES, 0:1]
                   + hist_ref[1, :N_NODES, 0:1])
            dinv = jax.lax.rsqrt(deg)
            h = jnp.maximum(
                dinv * p + dinv * dinv * xw_ref[...] + b_ref[...], 0.0)
            h_sc[:N_NODES, :] = h.astype(jnp.bfloat16)
            h_sc[N_NODES:, :] = jnp.zeros((N_PAD - N_NODES, DIM), jnp.bfloat16)

        a = h_sc[pl.ds(i * bm, bm), :]
        b = h_sc[pl.ds(j * bn, bn), :]
        o_ref[...] = jax.lax.dot_general(
            a, b, (((1,), (1,)), ((), ())),
            preferred_element_type=jnp.float32)

    return pl.pallas_call(
        body,
        grid=(gi, gj),
        in_specs=[
            pl.BlockSpec((NUM_CORES, N_PAD, DIM), lambda i, j: (0, 0, 0)),
            pl.BlockSpec((NUM_CORES, N_PAD, 16), lambda i, j: (0, 0, 0)),
            pl.BlockSpec((N_NODES, DIM), lambda i, j: (0, 0)),
            pl.BlockSpec((1, DIM), lambda i, j: (0, 0)),
        ],
        out_specs=pl.BlockSpec((bm, bn), lambda i, j: (i, j)),
        out_shape=jax.ShapeDtypeStruct((N_NODES, N_NODES), jnp.float32),
        scratch_shapes=[pltpu.VMEM((N_PAD, DIM), jnp.bfloat16)],
    )(part, hist, xw, b2)


def kernel(z, edge_index, W, b):
    ei3 = edge_index.astype(jnp.int32).reshape(2, -1, CHUNK)

    ones_pay = jnp.ones((CHUNK, 16), jnp.float32)
    zeros16 = jnp.zeros((ROWS_PER_SUB, 16), jnp.float32)
    zeros64 = jnp.zeros((ROWS_PER_SUB, DIM), jnp.float32)

    hist = _sc_degree(ei3, ones_pay, zeros16)
    xw, u = _tc_xw_u(z, W, hist)

    part = _sc_scatter(u, ei3, zeros64)
    return _tc_gram_fused(part, hist, xw, b.reshape(1, DIM))
